# baseline (speedup 1.0000x reference)
import functools

import jax
import jax.numpy as jnp
from jax import lax
from jax.experimental import pallas as pl
from jax.experimental.pallas import tpu as pltpu
from jax.experimental.pallas import tpu_sc as plsc

N_NODES = 10000
N_EDGES = 320000
NODE_DIM = 128
EDGE_DIM = 16
HIDDEN = 64
AW = 80

NC, NS = 2, 16
NW = NC * NS
EPW = N_EDGES // NW
CHUNK = 80
NCHUNK = EPW // CHUNK
NSTAGE = 10
RPS = N_NODES // NSTAGE


def _p_body(x_ref, w1_ref, p_ref):
    p_ref[...] = jnp.dot(x_ref[...], w1_ref[:NODE_DIM],
                         preferred_element_type=jnp.float32)


def _compute_p(x, w1):
    return pl.pallas_call(
        _p_body,
        out_shape=jax.ShapeDtypeStruct((N_NODES, HIDDEN), jnp.float32),
    )(x, w1)


_QB = 10000


def _q_body(e_ref, w1_ref, b1_ref, q_ref):
    q_ref[...] = (jnp.dot(e_ref[...], w1_ref[NODE_DIM:],
                          preferred_element_type=jnp.float32)
                  + b1_ref[...])


def _compute_q(e, w1, b1):
    return pl.pallas_call(
        _q_body,
        grid=(N_EDGES // _QB,),
        in_specs=[
            pl.BlockSpec((_QB, EDGE_DIM), lambda i: (i, 0)),
            pl.BlockSpec((NODE_DIM + EDGE_DIM, HIDDEN), lambda i: (0, 0)),
            pl.BlockSpec((1, HIDDEN), lambda i: (0, 0)),
        ],
        out_specs=pl.BlockSpec((_QB, HIDDEN), lambda i: (i, 0)),
        out_shape=jax.ShapeDtypeStruct((N_EDGES, HIDDEN), jnp.float32),
    )(e, w1, b1.reshape(1, HIDDEN))


def _sc_edge_body(p_hbm, q_hbm, w_hbm, src_hbm, dst_hbm, z_hbm, a_out0, a_out1,
                  src_buf, dst_buf, w_buf, q_buf, p_buf, m_buf, a_sh,
                  s_src, s_rest, s_g, s_sc):
    c = lax.axis_index("c")
    s = lax.axis_index("s")
    wid = s * NC + c
    r0 = pl.multiple_of(s * RPS, 8)

    @pl.when(s < NSTAGE)
    def _stage():
        pltpu.sync_copy(z_hbm.at[pl.ds(r0, RPS), :], a_sh.at[pl.ds(r0, RPS), :])

    plsc.subcore_barrier()

    def _base(ci):
        return pl.multiple_of(wid * EPW + ci * CHUNK, 8)

    def _issue_lin(b, ci):
        base = _base(ci)
        pltpu.async_copy(src_hbm.at[pl.ds(base, CHUNK)], src_buf.at[b],
                         s_src.at[b])
        pltpu.async_copy(dst_hbm.at[pl.ds(base, CHUNK)], dst_buf.at[b],
                         s_rest.at[b])
        pltpu.async_copy(w_hbm.at[pl.ds(base, CHUNK)], w_buf.at[b],
                         s_rest.at[b])
        pltpu.async_copy(q_hbm.at[pl.ds(base, CHUNK), :], q_buf.at[b],
                         s_rest.at[b])

    _issue_lin(0, 0)
    pltpu.make_async_copy(src_hbm.at[pl.ds(_base(0), CHUNK)], src_buf.at[0],
                          s_src.at[0]).wait()
    pltpu.async_copy(p_hbm.at[src_buf.at[0]], p_buf.at[0], s_g.at[0])

    def process(b, ci, issue_next):
        nb = 1 - b
        base = _base(ci)

        if issue_next:
            @pl.when(ci >= 1)
            def _protect():
                pltpu.make_async_copy(m_buf.at[nb], a_sh.at[dst_buf.at[nb]],
                                      s_sc.at[nb]).wait()

            _issue_lin(nb, ci + 1)

        pltpu.make_async_copy(dst_hbm.at[pl.ds(base, CHUNK)], dst_buf.at[b],
                              s_rest.at[b]).wait()
        pltpu.make_async_copy(w_hbm.at[pl.ds(base, CHUNK)], w_buf.at[b],
                              s_rest.at[b]).wait()
        pltpu.make_async_copy(q_hbm.at[pl.ds(base, CHUNK), :], q_buf.at[b],
                              s_rest.at[b]).wait()
        pltpu.make_async_copy(p_hbm.at[src_buf.at[b]], p_buf.at[b],
                              s_g.at[b]).wait()

        for grp in range(CHUNK // 16):
            w16 = w_buf[b, pl.ds(grp * 16, 16)]
            for j in range(16):
                g = grp * 16 + j
                wj = w16[j]
                for k in range(HIDDEN // 16):
                    z = (p_buf[b, g, pl.ds(16 * k, 16)]
                         + q_buf[b, g, pl.ds(16 * k, 16)])
                    h = jnp.maximum(z, 0.1 * z)
                    m_buf[b, g, pl.ds(16 * k, 16)] = wj * h
                m_buf[b, g, pl.ds(HIDDEN, 16)] = jnp.full((16,), wj, jnp.float32)
        pltpu.async_copy(m_buf.at[b], a_sh.at[dst_buf.at[b]], s_sc.at[b],
                         add=True)
        if issue_next:
            pltpu.make_async_copy(src_hbm.at[pl.ds(_base(ci + 1), CHUNK)],
                                  src_buf.at[nb], s_src.at[nb]).wait()
            pltpu.async_copy(p_hbm.at[src_buf.at[nb]], p_buf.at[nb],
                             s_g.at[nb])

    def pair(i, carry):
        ci = i * 2
        process(0, ci, True)
        process(1, ci + 1, True)
        return carry

    lax.fori_loop(0, (NCHUNK - 1) // 2, pair, 0)
    process(0, NCHUNK - 1, False)
    pltpu.make_async_copy(m_buf.at[0], a_sh.at[dst_buf.at[0]], s_sc.at[0]).wait()
    pltpu.make_async_copy(m_buf.at[1], a_sh.at[dst_buf.at[1]], s_sc.at[1]).wait()
    plsc.subcore_barrier()

    @pl.when((s < NSTAGE) & (c == 0))
    def _drain0():
        pltpu.sync_copy(a_sh.at[pl.ds(r0, RPS), :],
                        a_out0.at[pl.ds(r0, RPS), :])

    @pl.when((s < NSTAGE) & (c == 1))
    def _drain1():
        pltpu.sync_copy(a_sh.at[pl.ds(r0, RPS), :],
                        a_out1.at[pl.ds(r0, RPS), :])


def _sc_edge(p, q, w, src, dst):
    zeros = jnp.zeros((N_NODES, AW), jnp.float32)
    mesh = plsc.VectorSubcoreMesh(core_axis_name="c", subcore_axis_name="s")
    f = pl.kernel(
        _sc_edge_body,
        out_type=(jax.ShapeDtypeStruct((N_NODES, AW), jnp.float32),
                  jax.ShapeDtypeStruct((N_NODES, AW), jnp.float32)),
        mesh=mesh,
        scratch_types=[
            pltpu.VMEM((2, CHUNK), jnp.int32),
            pltpu.VMEM((2, CHUNK), jnp.int32),
            pltpu.VMEM((2, CHUNK), jnp.float32),
            pltpu.VMEM((2, CHUNK, HIDDEN), jnp.float32),
            pltpu.VMEM((2, CHUNK, HIDDEN), jnp.float32),
            pltpu.VMEM((2, CHUNK, AW), jnp.float32),
            pltpu.VMEM_SHARED((N_NODES, AW), jnp.float32),
            pltpu.SemaphoreType.DMA((2,)),
            pltpu.SemaphoreType.DMA((2,)),
            pltpu.SemaphoreType.DMA((2,)),
            pltpu.SemaphoreType.DMA((2,)),
        ],
        compiler_params=pltpu.CompilerParams(use_tc_tiling_on_sc=False),
    )
    return f(p, q, w, src, dst, zeros)


def _post_body(x_ref, a0_ref, a1_ref, w2_ref, b2_ref, w3_ref, b3_ref, g_ref,
               be_ref, w4_ref, b4_ref, o_ref):
    x = x_ref[...]
    a = a0_ref[...] + a1_ref[...]
    a64 = a[:, :HIDDEN]
    s16 = a[:, HIDDEN:]
    w3x = w3_ref[:NODE_DIM]
    w3a = w3_ref[NODE_DIM:]
    m = jnp.dot(w2_ref[...], w3a, preferred_element_type=jnp.float32)
    v = jnp.dot(b2_ref[...], w3a, preferred_element_type=jnp.float32)
    v_ext = jnp.concatenate([v, jnp.zeros((15, HIDDEN), jnp.float32)], axis=0)
    u = (jnp.dot(x, w3x, preferred_element_type=jnp.float32)
         + jnp.dot(a64, m, preferred_element_type=jnp.float32)
         + jnp.dot(s16, v_ext, preferred_element_type=jnp.float32)
         + b3_ref[...])
    mean = jnp.mean(u, axis=-1, keepdims=True)
    var = jnp.mean((u - mean) ** 2, axis=-1, keepdims=True)
    u = (u - mean) / jnp.sqrt(var + 1e-5) * g_ref[...] + be_ref[...]
    u = jnp.maximum(u, 0.1 * u)
    o_ref[...] = jnp.dot(u, w4_ref[...], preferred_element_type=jnp.float32) + b4_ref[...]


def _post(x, a0, a1, w2, b2, w3, b3, ln_g, ln_b, w4, b4):
    return pl.pallas_call(
        _post_body,
        out_shape=jax.ShapeDtypeStruct((N_NODES, NODE_DIM), jnp.float32),
    )(x, a0, a1, w2, b2.reshape(1, NODE_DIM), w3, b3.reshape(1, HIDDEN),
      ln_g.reshape(1, HIDDEN), ln_b.reshape(1, HIDDEN), w4,
      b4.reshape(1, NODE_DIM))


def kernel(node_features, edge_index, edge_attr, edge_weights,
           W1, b1, W2, b2, W3, b3, ln_g, ln_b, W4, b4):
    src = edge_index[0].astype(jnp.int32)
    dst = edge_index[1].astype(jnp.int32)
    p = _compute_p(node_features, W1)
    q = _compute_q(edge_attr, W1, b1)
    a0, a1 = _sc_edge(p, q, edge_weights, src, dst)
    return _post(node_features, a0, a1, W2, b2, W3, b3, ln_g, ln_b, W4, b4)


# speedup vs baseline: 1.2453x; 1.2453x over previous
import functools

import jax
import jax.numpy as jnp
from jax import lax
from jax.experimental import pallas as pl
from jax.experimental.pallas import tpu as pltpu
from jax.experimental.pallas import tpu_sc as plsc

N_NODES = 10000
N_EDGES = 320000
NODE_DIM = 128
EDGE_DIM = 16
HIDDEN = 64
AW = 80

NC, NS = 2, 16
NW = NC * NS
EPW = N_EDGES // NW
CHUNK = 80
NCHUNK = EPW // CHUNK
NSTAGE = 10
RPS = N_NODES // NSTAGE


def _p_body(x_ref, w1_ref, p_ref):
    p_ref[...] = jnp.dot(x_ref[...], w1_ref[:NODE_DIM],
                         preferred_element_type=jnp.float32)


def _compute_p(x, w1):
    return pl.pallas_call(
        _p_body,
        out_shape=jax.ShapeDtypeStruct((N_NODES, HIDDEN), jnp.float32),
    )(x, w1)


_QB = 20000


def _q_body(e2_ref, wbig_ref, bbig_ref, q_ref):
    q_ref[...] = (jnp.dot(e2_ref[...], wbig_ref[...],
                          preferred_element_type=jnp.float32)
                  + bbig_ref[...])


def _compute_q(e, w1, b1):
    e2 = e.reshape(N_EDGES // 2, 2 * EDGE_DIM)
    w1e = w1[NODE_DIM:]
    wbig = jnp.zeros((2 * EDGE_DIM, 2 * HIDDEN), jnp.float32)
    wbig = wbig.at[:EDGE_DIM, :HIDDEN].set(w1e).at[EDGE_DIM:, HIDDEN:].set(w1e)
    bbig = jnp.concatenate([b1, b1]).reshape(1, 2 * HIDDEN)
    return pl.pallas_call(
        _q_body,
        grid=(N_EDGES // 2 // _QB,),
        in_specs=[
            pl.BlockSpec((_QB, 2 * EDGE_DIM), lambda i: (i, 0)),
            pl.BlockSpec((2 * EDGE_DIM, 2 * HIDDEN), lambda i: (0, 0)),
            pl.BlockSpec((1, 2 * HIDDEN), lambda i: (0, 0)),
        ],
        out_specs=pl.BlockSpec((_QB, 2 * HIDDEN), lambda i: (i, 0)),
        out_shape=jax.ShapeDtypeStruct((N_EDGES // 2, 2 * HIDDEN), jnp.float32),
    )(e2, wbig, bbig)


def _sc_edge_body(p_hbm, q_hbm, w_hbm, src_hbm, dst_hbm, z_hbm, a_out0, a_out1,
                  src_buf, dst_buf, w_buf, q_buf, p_buf, m_buf, a_sh,
                  s_src, s_rest, s_g, s_sc):
    c = lax.axis_index("c")
    s = lax.axis_index("s")
    wid = s * NC + c
    r0 = pl.multiple_of(s * RPS, 8)

    @pl.when(s < NSTAGE)
    def _stage():
        pltpu.sync_copy(z_hbm.at[pl.ds(r0, RPS), :], a_sh.at[pl.ds(r0, RPS), :])

    plsc.subcore_barrier()

    def _base(ci):
        return pl.multiple_of(wid * EPW + ci * CHUNK, 8)

    def _base2(ci):
        return pl.multiple_of((wid * EPW) // 2 + ci * (CHUNK // 2), 8)

    def _issue_lin(b, ci):
        base = _base(ci)
        pltpu.async_copy(src_hbm.at[pl.ds(base, CHUNK)], src_buf.at[b],
                         s_src.at[b])
        pltpu.async_copy(dst_hbm.at[pl.ds(base, CHUNK)], dst_buf.at[b],
                         s_rest.at[b])
        pltpu.async_copy(w_hbm.at[pl.ds(base, CHUNK)], w_buf.at[b],
                         s_rest.at[b])
        pltpu.async_copy(q_hbm.at[pl.ds(_base2(ci), CHUNK // 2), :],
                         q_buf.at[b], s_rest.at[b])

    _issue_lin(0, 0)
    pltpu.make_async_copy(src_hbm.at[pl.ds(_base(0), CHUNK)], src_buf.at[0],
                          s_src.at[0]).wait()
    pltpu.async_copy(p_hbm.at[src_buf.at[0]], p_buf.at[0], s_g.at[0])

    def process(b, ci, issue_next):
        nb = 1 - b
        base = _base(ci)

        if issue_next:
            @pl.when(ci >= 1)
            def _protect():
                pltpu.make_async_copy(m_buf.at[nb], a_sh.at[dst_buf.at[nb]],
                                      s_sc.at[nb]).wait()

            _issue_lin(nb, ci + 1)

        pltpu.make_async_copy(dst_hbm.at[pl.ds(base, CHUNK)], dst_buf.at[b],
                              s_rest.at[b]).wait()
        pltpu.make_async_copy(w_hbm.at[pl.ds(base, CHUNK)], w_buf.at[b],
                              s_rest.at[b]).wait()
        pltpu.make_async_copy(q_hbm.at[pl.ds(_base2(ci), CHUNK // 2), :],
                              q_buf.at[b], s_rest.at[b]).wait()
        pltpu.make_async_copy(p_hbm.at[src_buf.at[b]], p_buf.at[b],
                              s_g.at[b]).wait()

        for grp in range(CHUNK // 16):
            w16 = w_buf[b, pl.ds(grp * 16, 16)]
            for j in range(16):
                g = grp * 16 + j
                wj = w16[j]
                for k in range(HIDDEN // 16):
                    z = (p_buf[b, g, pl.ds(16 * k, 16)]
                         + q_buf[b, g // 2, pl.ds(HIDDEN * (g % 2) + 16 * k, 16)])
                    h = jnp.maximum(z, 0.1 * z)
                    m_buf[b, g, pl.ds(16 * k, 16)] = wj * h
                m_buf[b, g, pl.ds(HIDDEN, 16)] = jnp.full((16,), wj, jnp.float32)
        pltpu.async_copy(m_buf.at[b], a_sh.at[dst_buf.at[b]], s_sc.at[b],
                         add=True)
        if issue_next:
            pltpu.make_async_copy(src_hbm.at[pl.ds(_base(ci + 1), CHUNK)],
                                  src_buf.at[nb], s_src.at[nb]).wait()
            pltpu.async_copy(p_hbm.at[src_buf.at[nb]], p_buf.at[nb],
                             s_g.at[nb])

    def pair(i, carry):
        ci = i * 2
        process(0, ci, True)
        process(1, ci + 1, True)
        return carry

    lax.fori_loop(0, (NCHUNK - 1) // 2, pair, 0)
    process(0, NCHUNK - 1, False)
    pltpu.make_async_copy(m_buf.at[0], a_sh.at[dst_buf.at[0]], s_sc.at[0]).wait()
    pltpu.make_async_copy(m_buf.at[1], a_sh.at[dst_buf.at[1]], s_sc.at[1]).wait()
    plsc.subcore_barrier()

    @pl.when((s < NSTAGE) & (c == 0))
    def _drain0():
        pltpu.sync_copy(a_sh.at[pl.ds(r0, RPS), :],
                        a_out0.at[pl.ds(r0, RPS), :])

    @pl.when((s < NSTAGE) & (c == 1))
    def _drain1():
        pltpu.sync_copy(a_sh.at[pl.ds(r0, RPS), :],
                        a_out1.at[pl.ds(r0, RPS), :])


def _sc_edge(p, q, w, src, dst):
    zeros = jnp.zeros((N_NODES, AW), jnp.float32)
    mesh = plsc.VectorSubcoreMesh(core_axis_name="c", subcore_axis_name="s")
    f = pl.kernel(
        _sc_edge_body,
        out_type=(jax.ShapeDtypeStruct((N_NODES, AW), jnp.float32),
                  jax.ShapeDtypeStruct((N_NODES, AW), jnp.float32)),
        mesh=mesh,
        scratch_types=[
            pltpu.VMEM((2, CHUNK), jnp.int32),
            pltpu.VMEM((2, CHUNK), jnp.int32),
            pltpu.VMEM((2, CHUNK), jnp.float32),
            pltpu.VMEM((2, CHUNK // 2, 2 * HIDDEN), jnp.float32),
            pltpu.VMEM((2, CHUNK, HIDDEN), jnp.float32),
            pltpu.VMEM((2, CHUNK, AW), jnp.float32),
            pltpu.VMEM_SHARED((N_NODES, AW), jnp.float32),
            pltpu.SemaphoreType.DMA((2,)),
            pltpu.SemaphoreType.DMA((2,)),
            pltpu.SemaphoreType.DMA((2,)),
            pltpu.SemaphoreType.DMA((2,)),
        ],
        compiler_params=pltpu.CompilerParams(use_tc_tiling_on_sc=False),
    )
    return f(p, q, w, src, dst, zeros)


def _post_body(x_ref, a0_ref, a1_ref, w2_ref, b2_ref, w3_ref, b3_ref, g_ref,
               be_ref, w4_ref, b4_ref, o_ref):
    x = x_ref[...]
    a = a0_ref[...] + a1_ref[...]
    a64 = a[:, :HIDDEN]
    s16 = a[:, HIDDEN:]
    w3x = w3_ref[:NODE_DIM]
    w3a = w3_ref[NODE_DIM:]
    m = jnp.dot(w2_ref[...], w3a, preferred_element_type=jnp.float32)
    v = jnp.dot(b2_ref[...], w3a, preferred_element_type=jnp.float32)
    v_ext = jnp.concatenate([v, jnp.zeros((15, HIDDEN), jnp.float32)], axis=0)
    u = (jnp.dot(x, w3x, preferred_element_type=jnp.float32)
         + jnp.dot(a64, m, preferred_element_type=jnp.float32)
         + jnp.dot(s16, v_ext, preferred_element_type=jnp.float32)
         + b3_ref[...])
    mean = jnp.mean(u, axis=-1, keepdims=True)
    var = jnp.mean((u - mean) ** 2, axis=-1, keepdims=True)
    u = (u - mean) / jnp.sqrt(var + 1e-5) * g_ref[...] + be_ref[...]
    u = jnp.maximum(u, 0.1 * u)
    o_ref[...] = jnp.dot(u, w4_ref[...], preferred_element_type=jnp.float32) + b4_ref[...]


def _post(x, a0, a1, w2, b2, w3, b3, ln_g, ln_b, w4, b4):
    return pl.pallas_call(
        _post_body,
        out_shape=jax.ShapeDtypeStruct((N_NODES, NODE_DIM), jnp.float32),
    )(x, a0, a1, w2, b2.reshape(1, NODE_DIM), w3, b3.reshape(1, HIDDEN),
      ln_g.reshape(1, HIDDEN), ln_b.reshape(1, HIDDEN), w4,
      b4.reshape(1, NODE_DIM))


def kernel(node_features, edge_index, edge_attr, edge_weights,
           W1, b1, W2, b2, W3, b3, ln_g, ln_b, W4, b4):
    src = edge_index[0].astype(jnp.int32)
    dst = edge_index[1].astype(jnp.int32)
    p = _compute_p(node_features, W1)
    q = _compute_q(edge_attr, W1, b1)
    a0, a1 = _sc_edge(p, q, edge_weights, src, dst)
    return _post(node_features, a0, a1, W2, b2, W3, b3, ln_g, ln_b, W4, b4)


# speedup vs baseline: 1.3120x; 1.0536x over previous
import functools

import jax
import jax.numpy as jnp
from jax import lax
from jax.experimental import pallas as pl
from jax.experimental.pallas import tpu as pltpu
from jax.experimental.pallas import tpu_sc as plsc

N_NODES = 10000
N_EDGES = 320000
NODE_DIM = 128
EDGE_DIM = 16
HIDDEN = 64
AW = 80

NC, NS = 2, 16
NW = NC * NS
EPW = N_EDGES // NW
CHUNK = 80
NCHUNK = EPW // CHUNK
NSTAGE = 10
RPS = N_NODES // NSTAGE


def _p_body(x_ref, w1_ref, p_ref):
    p_ref[...] = jnp.dot(x_ref[...], w1_ref[:NODE_DIM],
                         preferred_element_type=jnp.float32)


def _compute_p(x, w1):
    return pl.pallas_call(
        _p_body,
        out_shape=jax.ShapeDtypeStruct((N_NODES, HIDDEN), jnp.float32),
    )(x, w1)


_QB = 10000


def _q_body(elo_ref, ehi_ref, w1_ref, b1_ref, q_ref):
    w1e = w1_ref[NODE_DIM:]
    b1 = b1_ref[...]
    qlo = jnp.dot(elo_ref[...], w1e, preferred_element_type=jnp.float32) + b1
    qhi = jnp.dot(ehi_ref[...], w1e, preferred_element_type=jnp.float32) + b1
    q_ref[...] = jnp.concatenate([qlo, qhi], axis=1)


def _compute_q(e, w1, b1):
    return pl.pallas_call(
        _q_body,
        grid=(N_EDGES // 2 // _QB,),
        in_specs=[
            pl.BlockSpec((_QB, EDGE_DIM), lambda i: (2 * i, 0)),
            pl.BlockSpec((_QB, EDGE_DIM), lambda i: (2 * i + 1, 0)),
            pl.BlockSpec((NODE_DIM + EDGE_DIM, HIDDEN), lambda i: (0, 0)),
            pl.BlockSpec((1, HIDDEN), lambda i: (0, 0)),
        ],
        out_specs=pl.BlockSpec((_QB, 2 * HIDDEN), lambda i: (i, 0)),
        out_shape=jax.ShapeDtypeStruct((N_EDGES // 2, 2 * HIDDEN), jnp.float32),
    )(e, e, w1, b1.reshape(1, HIDDEN))


def _sc_edge_body(p_hbm, q_hbm, w_hbm, src_hbm, dst_hbm, z_hbm, a_out0, a_out1,
                  src_buf, dst_buf, w_buf, q_buf, p_buf, m_buf, a_sh,
                  s_src, s_rest, s_g, s_sc):
    c = lax.axis_index("c")
    s = lax.axis_index("s")
    wid = s * NC + c
    r0 = pl.multiple_of(s * RPS, 8)

    @pl.when(s < NSTAGE)
    def _stage():
        pltpu.sync_copy(z_hbm.at[pl.ds(r0, RPS), :], a_sh.at[pl.ds(r0, RPS), :])

    plsc.subcore_barrier()

    HALF = CHUNK // 2
    i_sb = wid // (NW // (N_EDGES // (2 * _QB)))

    def _pr0(ci):
        return pl.multiple_of((wid * EPW) // 2 + ci * HALF, 8)

    def _lo0(ci):
        return pl.multiple_of(_pr0(ci) + i_sb * _QB, 8)

    def _issue_lin(b, ci):
        lo0 = _lo0(ci)
        hi0 = pl.multiple_of(lo0 + _QB, 8)
        for a_hbm, buf, sem in ((src_hbm, src_buf, s_src),
                                (dst_hbm, dst_buf, s_rest),
                                (w_hbm, w_buf, s_rest)):
            pltpu.async_copy(a_hbm.at[pl.ds(lo0, HALF)],
                             buf.at[b, pl.ds(0, HALF)], sem.at[b])
            pltpu.async_copy(a_hbm.at[pl.ds(hi0, HALF)],
                             buf.at[b, pl.ds(HALF, HALF)], sem.at[b])
        pltpu.async_copy(q_hbm.at[pl.ds(_pr0(ci), HALF), :],
                         q_buf.at[b], s_rest.at[b])

    def _wait_src(b, ci):
        lo0 = _lo0(ci)
        hi0 = pl.multiple_of(lo0 + _QB, 8)
        pltpu.make_async_copy(src_hbm.at[pl.ds(lo0, HALF)],
                              src_buf.at[b, pl.ds(0, HALF)], s_src.at[b]).wait()
        pltpu.make_async_copy(src_hbm.at[pl.ds(hi0, HALF)],
                              src_buf.at[b, pl.ds(HALF, HALF)],
                              s_src.at[b]).wait()

    def _wait_rest(b, ci):
        lo0 = _lo0(ci)
        hi0 = pl.multiple_of(lo0 + _QB, 8)
        for a_hbm, buf in ((dst_hbm, dst_buf), (w_hbm, w_buf)):
            pltpu.make_async_copy(a_hbm.at[pl.ds(lo0, HALF)],
                                  buf.at[b, pl.ds(0, HALF)],
                                  s_rest.at[b]).wait()
            pltpu.make_async_copy(a_hbm.at[pl.ds(hi0, HALF)],
                                  buf.at[b, pl.ds(HALF, HALF)],
                                  s_rest.at[b]).wait()
        pltpu.make_async_copy(q_hbm.at[pl.ds(_pr0(ci), HALF), :],
                              q_buf.at[b], s_rest.at[b]).wait()

    _issue_lin(0, 0)
    _wait_src(0, 0)
    pltpu.async_copy(p_hbm.at[src_buf.at[0]], p_buf.at[0], s_g.at[0])

    def process(b, ci, issue_next):
        nb = 1 - b

        if issue_next:
            @pl.when(ci >= 1)
            def _protect():
                pltpu.make_async_copy(m_buf.at[nb], a_sh.at[dst_buf.at[nb]],
                                      s_sc.at[nb]).wait()

            _issue_lin(nb, ci + 1)

        _wait_rest(b, ci)
        pltpu.make_async_copy(p_hbm.at[src_buf.at[b]], p_buf.at[b],
                              s_g.at[b]).wait()

        for grp in range(CHUNK // 16):
            w16 = w_buf[b, pl.ds(grp * 16, 16)]
            for j in range(16):
                g = grp * 16 + j
                wj = w16[j]
                qrow = g if g < HALF else g - HALF
                qcol = 0 if g < HALF else HIDDEN
                for k in range(HIDDEN // 16):
                    z = (p_buf[b, g, pl.ds(16 * k, 16)]
                         + q_buf[b, qrow, pl.ds(qcol + 16 * k, 16)])
                    h = jnp.maximum(z, 0.1 * z)
                    m_buf[b, g, pl.ds(16 * k, 16)] = wj * h
                m_buf[b, g, pl.ds(HIDDEN, 16)] = jnp.full((16,), wj, jnp.float32)
        pltpu.async_copy(m_buf.at[b], a_sh.at[dst_buf.at[b]], s_sc.at[b],
                         add=True)
        if issue_next:
            _wait_src(nb, ci + 1)
            pltpu.async_copy(p_hbm.at[src_buf.at[nb]], p_buf.at[nb],
                             s_g.at[nb])

    def pair(i, carry):
        ci = i * 2
        process(0, ci, True)
        process(1, ci + 1, True)
        return carry

    lax.fori_loop(0, (NCHUNK - 1) // 2, pair, 0)
    process(0, NCHUNK - 1, False)
    pltpu.make_async_copy(m_buf.at[0], a_sh.at[dst_buf.at[0]], s_sc.at[0]).wait()
    pltpu.make_async_copy(m_buf.at[1], a_sh.at[dst_buf.at[1]], s_sc.at[1]).wait()
    plsc.subcore_barrier()

    @pl.when((s < NSTAGE) & (c == 0))
    def _drain0():
        pltpu.sync_copy(a_sh.at[pl.ds(r0, RPS), :],
                        a_out0.at[pl.ds(r0, RPS), :])

    @pl.when((s < NSTAGE) & (c == 1))
    def _drain1():
        pltpu.sync_copy(a_sh.at[pl.ds(r0, RPS), :],
                        a_out1.at[pl.ds(r0, RPS), :])


def _sc_edge(p, q, w, src, dst):
    zeros = jnp.zeros((N_NODES, AW), jnp.float32)
    mesh = plsc.VectorSubcoreMesh(core_axis_name="c", subcore_axis_name="s")
    f = pl.kernel(
        _sc_edge_body,
        out_type=(jax.ShapeDtypeStruct((N_NODES, AW), jnp.float32),
                  jax.ShapeDtypeStruct((N_NODES, AW), jnp.float32)),
        mesh=mesh,
        scratch_types=[
            pltpu.VMEM((2, CHUNK), jnp.int32),
            pltpu.VMEM((2, CHUNK), jnp.int32),
            pltpu.VMEM((2, CHUNK), jnp.float32),
            pltpu.VMEM((2, CHUNK // 2, 2 * HIDDEN), jnp.float32),
            pltpu.VMEM((2, CHUNK, HIDDEN), jnp.float32),
            pltpu.VMEM((2, CHUNK, AW), jnp.float32),
            pltpu.VMEM_SHARED((N_NODES, AW), jnp.float32),
            pltpu.SemaphoreType.DMA((2,)),
            pltpu.SemaphoreType.DMA((2,)),
            pltpu.SemaphoreType.DMA((2,)),
            pltpu.SemaphoreType.DMA((2,)),
        ],
        compiler_params=pltpu.CompilerParams(use_tc_tiling_on_sc=False),
    )
    return f(p, q, w, src, dst, zeros)


def _post_body(x_ref, a0_ref, a1_ref, w2_ref, b2_ref, w3_ref, b3_ref, g_ref,
               be_ref, w4_ref, b4_ref, o_ref):
    x = x_ref[...]
    a = a0_ref[...] + a1_ref[...]
    a64 = a[:, :HIDDEN]
    s16 = a[:, HIDDEN:]
    w3x = w3_ref[:NODE_DIM]
    w3a = w3_ref[NODE_DIM:]
    m = jnp.dot(w2_ref[...], w3a, preferred_element_type=jnp.float32)
    v = jnp.dot(b2_ref[...], w3a, preferred_element_type=jnp.float32)
    v_ext = jnp.concatenate([v, jnp.zeros((15, HIDDEN), jnp.float32)], axis=0)
    u = (jnp.dot(x, w3x, preferred_element_type=jnp.float32)
         + jnp.dot(a64, m, preferred_element_type=jnp.float32)
         + jnp.dot(s16, v_ext, preferred_element_type=jnp.float32)
         + b3_ref[...])
    mean = jnp.mean(u, axis=-1, keepdims=True)
    var = jnp.mean((u - mean) ** 2, axis=-1, keepdims=True)
    u = (u - mean) / jnp.sqrt(var + 1e-5) * g_ref[...] + be_ref[...]
    u = jnp.maximum(u, 0.1 * u)
    o_ref[...] = jnp.dot(u, w4_ref[...], preferred_element_type=jnp.float32) + b4_ref[...]


def _post(x, a0, a1, w2, b2, w3, b3, ln_g, ln_b, w4, b4):
    return pl.pallas_call(
        _post_body,
        out_shape=jax.ShapeDtypeStruct((N_NODES, NODE_DIM), jnp.float32),
    )(x, a0, a1, w2, b2.reshape(1, NODE_DIM), w3, b3.reshape(1, HIDDEN),
      ln_g.reshape(1, HIDDEN), ln_b.reshape(1, HIDDEN), w4,
      b4.reshape(1, NODE_DIM))


def kernel(node_features, edge_index, edge_attr, edge_weights,
           W1, b1, W2, b2, W3, b3, ln_g, ln_b, W4, b4):
    src = edge_index[0].astype(jnp.int32)
    dst = edge_index[1].astype(jnp.int32)
    p = _compute_p(node_features, W1)
    q = _compute_q(edge_attr, W1, b1)
    a0, a1 = _sc_edge(p, q, edge_weights, src, dst)
    return _post(node_features, a0, a1, W2, b2, W3, b3, ln_g, ln_b, W4, b4)


# speedup vs baseline: 1.3130x; 1.0008x over previous
import functools

import jax
import jax.numpy as jnp
from jax import lax
from jax.experimental import pallas as pl
from jax.experimental.pallas import tpu as pltpu
from jax.experimental.pallas import tpu_sc as plsc

N_NODES = 10000
N_EDGES = 320000
NODE_DIM = 128
EDGE_DIM = 16
HIDDEN = 64
AW = 80

NC, NS = 2, 16
NW = NC * NS
EPW = N_EDGES // NW
CHUNK = 80
NCHUNK = EPW // CHUNK
NSTAGE = 10
RPS = N_NODES // NSTAGE


def _p_body(x_ref, w1_ref, p_ref):
    p_ref[...] = jnp.dot(x_ref[...], w1_ref[:NODE_DIM],
                         preferred_element_type=jnp.float32)


def _compute_p(x, w1):
    return pl.pallas_call(
        _p_body,
        out_shape=jax.ShapeDtypeStruct((N_NODES, HIDDEN), jnp.float32),
    )(x, w1)


_QB = 20000


def _q_body(elo_ref, ehi_ref, w1_ref, b1_ref, q_ref):
    w1e = w1_ref[NODE_DIM:]
    b1 = b1_ref[...]
    qlo = jnp.dot(elo_ref[...], w1e, preferred_element_type=jnp.float32) + b1
    qhi = jnp.dot(ehi_ref[...], w1e, preferred_element_type=jnp.float32) + b1
    q_ref[...] = jnp.concatenate([qlo, qhi], axis=1)


def _compute_q(e, w1, b1):
    return pl.pallas_call(
        _q_body,
        grid=(N_EDGES // 2 // _QB,),
        in_specs=[
            pl.BlockSpec((_QB, EDGE_DIM), lambda i: (2 * i, 0)),
            pl.BlockSpec((_QB, EDGE_DIM), lambda i: (2 * i + 1, 0)),
            pl.BlockSpec((NODE_DIM + EDGE_DIM, HIDDEN), lambda i: (0, 0)),
            pl.BlockSpec((1, HIDDEN), lambda i: (0, 0)),
        ],
        out_specs=pl.BlockSpec((_QB, 2 * HIDDEN), lambda i: (i, 0)),
        out_shape=jax.ShapeDtypeStruct((N_EDGES // 2, 2 * HIDDEN), jnp.float32),
        compiler_params=pltpu.CompilerParams(vmem_limit_bytes=100 * 1024 * 1024),
    )(e, e, w1, b1.reshape(1, HIDDEN))


def _sc_edge_body(p_hbm, q_hbm, w_hbm, src_hbm, dst_hbm, z_hbm, a_out0, a_out1,
                  src_buf, dst_buf, w_buf, q_buf, p_buf, m_buf, a_sh,
                  s_src, s_rest, s_g, s_sc):
    c = lax.axis_index("c")
    s = lax.axis_index("s")
    wid = s * NC + c
    r0 = pl.multiple_of(s * RPS, 8)

    @pl.when(s < NSTAGE)
    def _stage():
        pltpu.sync_copy(z_hbm.at[pl.ds(r0, RPS), :], a_sh.at[pl.ds(r0, RPS), :])

    plsc.subcore_barrier()

    HALF = CHUNK // 2
    i_sb = wid // (NW // (N_EDGES // (2 * _QB)))

    def _pr0(ci):
        return pl.multiple_of((wid * EPW) // 2 + ci * HALF, 8)

    def _lo0(ci):
        return pl.multiple_of(_pr0(ci) + i_sb * _QB, 8)

    def _issue_lin(b, ci):
        lo0 = _lo0(ci)
        hi0 = pl.multiple_of(lo0 + _QB, 8)
        for a_hbm, buf, sem in ((src_hbm, src_buf, s_src),
                                (dst_hbm, dst_buf, s_rest),
                                (w_hbm, w_buf, s_rest)):
            pltpu.async_copy(a_hbm.at[pl.ds(lo0, HALF)],
                             buf.at[b, pl.ds(0, HALF)], sem.at[b])
            pltpu.async_copy(a_hbm.at[pl.ds(hi0, HALF)],
                             buf.at[b, pl.ds(HALF, HALF)], sem.at[b])
        pltpu.async_copy(q_hbm.at[pl.ds(_pr0(ci), HALF), :],
                         q_buf.at[b], s_rest.at[b])

    def _wait_src(b, ci):
        lo0 = _lo0(ci)
        hi0 = pl.multiple_of(lo0 + _QB, 8)
        pltpu.make_async_copy(src_hbm.at[pl.ds(lo0, HALF)],
                              src_buf.at[b, pl.ds(0, HALF)], s_src.at[b]).wait()
        pltpu.make_async_copy(src_hbm.at[pl.ds(hi0, HALF)],
                              src_buf.at[b, pl.ds(HALF, HALF)],
                              s_src.at[b]).wait()

    def _wait_rest(b, ci):
        lo0 = _lo0(ci)
        hi0 = pl.multiple_of(lo0 + _QB, 8)
        for a_hbm, buf in ((dst_hbm, dst_buf), (w_hbm, w_buf)):
            pltpu.make_async_copy(a_hbm.at[pl.ds(lo0, HALF)],
                                  buf.at[b, pl.ds(0, HALF)],
                                  s_rest.at[b]).wait()
            pltpu.make_async_copy(a_hbm.at[pl.ds(hi0, HALF)],
                                  buf.at[b, pl.ds(HALF, HALF)],
                                  s_rest.at[b]).wait()
        pltpu.make_async_copy(q_hbm.at[pl.ds(_pr0(ci), HALF), :],
                              q_buf.at[b], s_rest.at[b]).wait()

    _issue_lin(0, 0)
    _wait_src(0, 0)
    pltpu.async_copy(p_hbm.at[src_buf.at[0]], p_buf.at[0], s_g.at[0])

    def process(b, ci, issue_next):
        nb = 1 - b

        if issue_next:
            @pl.when(ci >= 1)
            def _protect():
                pltpu.make_async_copy(m_buf.at[nb], a_sh.at[dst_buf.at[nb]],
                                      s_sc.at[nb]).wait()

            _issue_lin(nb, ci + 1)

        _wait_rest(b, ci)
        pltpu.make_async_copy(p_hbm.at[src_buf.at[b]], p_buf.at[b],
                              s_g.at[b]).wait()

        for grp in range(CHUNK // 16):
            w16 = w_buf[b, pl.ds(grp * 16, 16)]
            for j in range(16):
                g = grp * 16 + j
                wj = w16[j]
                qrow = g if g < HALF else g - HALF
                qcol = 0 if g < HALF else HIDDEN
                for k in range(HIDDEN // 16):
                    z = (p_buf[b, g, pl.ds(16 * k, 16)]
                         + q_buf[b, qrow, pl.ds(qcol + 16 * k, 16)])
                    h = jnp.maximum(z, 0.1 * z)
                    m_buf[b, g, pl.ds(16 * k, 16)] = wj * h
                m_buf[b, g, pl.ds(HIDDEN, 16)] = jnp.full((16,), wj, jnp.float32)
        pltpu.async_copy(m_buf.at[b], a_sh.at[dst_buf.at[b]], s_sc.at[b],
                         add=True)
        if issue_next:
            _wait_src(nb, ci + 1)
            pltpu.async_copy(p_hbm.at[src_buf.at[nb]], p_buf.at[nb],
                             s_g.at[nb])

    def pair(i, carry):
        ci = i * 2
        process(0, ci, True)
        process(1, ci + 1, True)
        return carry

    lax.fori_loop(0, (NCHUNK - 1) // 2, pair, 0)
    process(0, NCHUNK - 1, False)
    pltpu.make_async_copy(m_buf.at[0], a_sh.at[dst_buf.at[0]], s_sc.at[0]).wait()
    pltpu.make_async_copy(m_buf.at[1], a_sh.at[dst_buf.at[1]], s_sc.at[1]).wait()
    plsc.subcore_barrier()

    @pl.when((s < NSTAGE) & (c == 0))
    def _drain0():
        pltpu.sync_copy(a_sh.at[pl.ds(r0, RPS), :],
                        a_out0.at[pl.ds(r0, RPS), :])

    @pl.when((s < NSTAGE) & (c == 1))
    def _drain1():
        pltpu.sync_copy(a_sh.at[pl.ds(r0, RPS), :],
                        a_out1.at[pl.ds(r0, RPS), :])


def _sc_edge(p, q, w, src, dst):
    zeros = jnp.zeros((N_NODES, AW), jnp.float32)
    mesh = plsc.VectorSubcoreMesh(core_axis_name="c", subcore_axis_name="s")
    f = pl.kernel(
        _sc_edge_body,
        out_type=(jax.ShapeDtypeStruct((N_NODES, AW), jnp.float32),
                  jax.ShapeDtypeStruct((N_NODES, AW), jnp.float32)),
        mesh=mesh,
        scratch_types=[
            pltpu.VMEM((2, CHUNK), jnp.int32),
            pltpu.VMEM((2, CHUNK), jnp.int32),
            pltpu.VMEM((2, CHUNK), jnp.float32),
            pltpu.VMEM((2, CHUNK // 2, 2 * HIDDEN), jnp.float32),
            pltpu.VMEM((2, CHUNK, HIDDEN), jnp.float32),
            pltpu.VMEM((2, CHUNK, AW), jnp.float32),
            pltpu.VMEM_SHARED((N_NODES, AW), jnp.float32),
            pltpu.SemaphoreType.DMA((2,)),
            pltpu.SemaphoreType.DMA((2,)),
            pltpu.SemaphoreType.DMA((2,)),
            pltpu.SemaphoreType.DMA((2,)),
        ],
        compiler_params=pltpu.CompilerParams(use_tc_tiling_on_sc=False),
    )
    return f(p, q, w, src, dst, zeros)


def _post_body(x_ref, a0_ref, a1_ref, w2_ref, b2_ref, w3_ref, b3_ref, g_ref,
               be_ref, w4_ref, b4_ref, o_ref):
    x = x_ref[...]
    a = a0_ref[...] + a1_ref[...]
    a64 = a[:, :HIDDEN]
    s16 = a[:, HIDDEN:]
    w3x = w3_ref[:NODE_DIM]
    w3a = w3_ref[NODE_DIM:]
    m = jnp.dot(w2_ref[...], w3a, preferred_element_type=jnp.float32)
    v = jnp.dot(b2_ref[...], w3a, preferred_element_type=jnp.float32)
    v_ext = jnp.concatenate([v, jnp.zeros((15, HIDDEN), jnp.float32)], axis=0)
    u = (jnp.dot(x, w3x, preferred_element_type=jnp.float32)
         + jnp.dot(a64, m, preferred_element_type=jnp.float32)
         + jnp.dot(s16, v_ext, preferred_element_type=jnp.float32)
         + b3_ref[...])
    mean = jnp.mean(u, axis=-1, keepdims=True)
    var = jnp.mean((u - mean) ** 2, axis=-1, keepdims=True)
    u = (u - mean) / jnp.sqrt(var + 1e-5) * g_ref[...] + be_ref[...]
    u = jnp.maximum(u, 0.1 * u)
    o_ref[...] = jnp.dot(u, w4_ref[...], preferred_element_type=jnp.float32) + b4_ref[...]


def _post(x, a0, a1, w2, b2, w3, b3, ln_g, ln_b, w4, b4):
    return pl.pallas_call(
        _post_body,
        out_shape=jax.ShapeDtypeStruct((N_NODES, NODE_DIM), jnp.float32),
    )(x, a0, a1, w2, b2.reshape(1, NODE_DIM), w3, b3.reshape(1, HIDDEN),
      ln_g.reshape(1, HIDDEN), ln_b.reshape(1, HIDDEN), w4,
      b4.reshape(1, NODE_DIM))


def kernel(node_features, edge_index, edge_attr, edge_weights,
           W1, b1, W2, b2, W3, b3, ln_g, ln_b, W4, b4):
    src = edge_index[0].astype(jnp.int32)
    dst = edge_index[1].astype(jnp.int32)
    p = _compute_p(node_features, W1)
    q = _compute_q(edge_attr, W1, b1)
    a0, a1 = _sc_edge(p, q, edge_weights, src, dst)
    return _post(node_features, a0, a1, W2, b2, W3, b3, ln_g, ln_b, W4, b4)
